# baseline (speedup 1.0000x reference)
import functools

import jax
import jax.numpy as jnp
from jax import lax
from jax.experimental import pallas as pl
from jax.experimental.pallas import tpu as pltpu
from jax.experimental.pallas import tpu_sc as plsc

_T = 64
_U = 16
_D = 1024
_NEG = -1e30


def _lse_body(x_ref, o_lse):
    x = x_ref[0]
    m = jnp.max(x, axis=2)
    e = jnp.exp(x - m[:, :, None])
    o_lse[:] = m + jnp.log(jnp.sum(e, axis=2))


def _tc_lse(logits):
    return pl.pallas_call(
        _lse_body,
        out_shape=jax.ShapeDtypeStruct((_T, _U), jnp.float32),
        in_specs=[pl.BlockSpec((1, _T, _U, _D), lambda: (0, 0, 0, 0))],
        out_specs=pl.BlockSpec((_T, _U), lambda: (0, 0)),
    )(logits)


_MESH = plsc.VectorSubcoreMesh(core_axis_name="c", subcore_axis_name="s")


@functools.partial(
    pl.kernel,
    mesh=_MESH,
    out_type=[
        jax.ShapeDtypeStruct((_T,), jnp.float32),
        jax.ShapeDtypeStruct((_T,), jnp.float32),
        jax.ShapeDtypeStruct((_T,), jnp.float32),
    ],
    compiler_params=pltpu.CompilerParams(needs_layout_passes=False),
    scratch_types=[
        pltpu.VMEM((_U,), jnp.int32),
        pltpu.VMEM((_U,), jnp.int32),
        pltpu.VMEM((_T * _U,), jnp.int32),
        pltpu.VMEM((_T * _U,), jnp.int32),
        pltpu.VMEM((_T * _U,), jnp.float32),
        pltpu.VMEM((_T * _U,), jnp.float32),
        pltpu.VMEM((_T * _U,), jnp.float32),
        pltpu.VMEM((_T, _U), jnp.float32),
        pltpu.VMEM((_T, _U), jnp.float32),
        pltpu.VMEM((_T, _U), jnp.float32),
        pltpu.VMEM((_U,), jnp.float32),
        pltpu.VMEM((_U,), jnp.float32),
        pltpu.VMEM((_T,), jnp.float32),
        pltpu.VMEM((_T,), jnp.float32),
        pltpu.VMEM((_T,), jnp.float32),
        pltpu.SemaphoreType.DMA,
    ],
)
def _sc_gather_dp(
    logits_hbm, lse_hbm, tgt_hbm, tl_hbm,
    o_la, o_st, o_tot,
    tgt_v, tl_v, idx_t, idx_b, graw, blraw, lse_v,
    laebuf, stbuf, totbuf, st_s, tot_s,
    stage_la, stage_st, stage_tot, sem,
):
    wid = lax.axis_index("s") * 2 + lax.axis_index("c")

    @pl.when(wid == 0)
    def _():
        lane = lax.iota(jnp.int32, _U)
        lanef = lane.astype(jnp.float32)

        pltpu.sync_copy(tgt_hbm, tgt_v)
        pltpu.sync_copy(tl_hbm, tl_v)
        pltpu.sync_copy(lse_hbm, lse_v)
        tgt_vec = tgt_v[...]
        tl_vec = tl_v[...]

        def build(i, _):
            r = (lane + i * _U) * _D
            idx_t[pl.ds(i * _U, _U)] = r + tgt_vec
            idx_b[pl.ds(i * _U, _U)] = r
            return 0

        lax.fori_loop(0, _T, build, 0, unroll=False)

        pltpu.async_copy(logits_hbm.at[idx_t], graw, sem).wait()
        pltpu.async_copy(logits_hbm.at[idx_b], blraw, sem).wait()

        def row(t, carry):
            la_p, st_p, tot_p, bl_p = carry
            lse_row = lse_v[pl.ds(t * _U, _U)]
            g_row = graw[pl.ds(t * _U, _U)] - lse_row
            bl_row = blraw[pl.ds(t * _U, _U)] - lse_row
            gi = plsc.cumsum(g_row)
            G = gi - g_row
            fl = la_p + bl_p
            fl0 = jnp.where(lane == 0, 0.0, fl)
            v = fl0 - G
            pm = plsc.cummax(v)
            la = G + pm
            rec = v >= pm
            jidx = jnp.where(rec, lane, -1)
            js = plsc.cummax(jidx)
            st_s[...] = st_p
            tot_s[...] = tot_p
            stg = plsc.load_gather(st_s, [js])
            totg = plsc.load_gather(tot_s, [js])
            tf = t.astype(jnp.float32)
            j0 = js == 0
            jsf = js.astype(jnp.float32)
            st = jnp.where(j0, tf, stg)
            tot = jnp.where(j0, 1.0, totg + 1.0) + (lanef - jsf)
            laebuf[t] = la + bl_row
            stbuf[t] = st
            totbuf[t] = tot
            return (la, st, tot, bl_row)

        init = (
            jnp.full((_U,), _NEG, jnp.float32),
            jnp.zeros((_U,), jnp.float32),
            jnp.zeros((_U,), jnp.float32),
            jnp.zeros((_U,), jnp.float32),
        )
        lax.fori_loop(0, _T, row, init, unroll=False)

        for c in range(_T // _U):
            rowidx = lane + c * _U
            stage_la[pl.ds(c * _U, _U)] = plsc.load_gather(laebuf, [rowidx, tl_vec])
            stage_st[pl.ds(c * _U, _U)] = plsc.load_gather(stbuf, [rowidx, tl_vec])
            stage_tot[pl.ds(c * _U, _U)] = plsc.load_gather(totbuf, [rowidx, tl_vec]) + 1.0

        pltpu.sync_copy(stage_la, o_la)
        pltpu.sync_copy(stage_st, o_st)
        pltpu.sync_copy(stage_tot, o_tot)


def kernel(logits, targets, logit_lens, target_lens):
    lse = _tc_lse(logits).reshape(_T * _U)
    logits_flat = logits.reshape(_T * _U * _D)
    tgt = targets.reshape(_U).astype(jnp.int32)
    tl16 = jnp.broadcast_to(target_lens.astype(jnp.int32), (_U,))
    la_each, st_each, tot_each = _sc_gather_dp(logits_flat, lse, tgt, tl16)
    return (la_each[_T - 1], la_each, st_each, tot_each)


# speedup vs baseline: 1.0128x; 1.0128x over previous
import functools

import jax
import jax.numpy as jnp
from jax import lax
from jax.experimental import pallas as pl
from jax.experimental.pallas import tpu as pltpu
from jax.experimental.pallas import tpu_sc as plsc

_T = 64
_U = 16
_D = 1024
_NEG = -1e30


def _lse_body(x_ref, o_lse):
    x = x_ref[0]
    m = jnp.max(x, axis=2)
    e = jnp.exp(x - m[:, :, None])
    o_lse[:] = m + jnp.log(jnp.sum(e, axis=2))


def _tc_lse(logits):
    return pl.pallas_call(
        _lse_body,
        out_shape=jax.ShapeDtypeStruct((_T, _U), jnp.float32),
        in_specs=[pl.BlockSpec((1, _T, _U, _D), lambda: (0, 0, 0, 0))],
        out_specs=pl.BlockSpec((_T, _U), lambda: (0, 0)),
    )(logits)


_MESH = plsc.VectorSubcoreMesh(core_axis_name="c", subcore_axis_name="s")


@functools.partial(
    pl.kernel,
    mesh=_MESH,
    out_type=[
        jax.ShapeDtypeStruct((_T,), jnp.float32),
        jax.ShapeDtypeStruct((_T,), jnp.float32),
        jax.ShapeDtypeStruct((_T,), jnp.float32),
    ],
    compiler_params=pltpu.CompilerParams(needs_layout_passes=False),
    scratch_types=[
        pltpu.VMEM((_U,), jnp.int32),
        pltpu.VMEM((_U,), jnp.int32),
        pltpu.VMEM((2 * _T * _U,), jnp.int32),
        pltpu.VMEM((2 * _T * _U,), jnp.float32),
        pltpu.VMEM((_T * _U,), jnp.float32),
        pltpu.VMEM((_T, _U), jnp.float32),
        pltpu.VMEM((_T, _U), jnp.float32),
        pltpu.VMEM((_T, _U), jnp.float32),
        pltpu.VMEM((_U,), jnp.float32),
        pltpu.VMEM((_U,), jnp.float32),
        pltpu.VMEM((_T,), jnp.float32),
        pltpu.VMEM((_T,), jnp.float32),
        pltpu.VMEM((_T,), jnp.float32),
        pltpu.SemaphoreType.DMA,
    ],
)
def _sc_gather_dp(
    logits_hbm, lse_hbm, tgt_hbm, tl_hbm,
    o_la, o_st, o_tot,
    tgt_v, tl_v, idx, raw, lse_v,
    laebuf, stbuf, totbuf, st_s, tot_s,
    stage_la, stage_st, stage_tot, sem,
):
    wid = lax.axis_index("s") * 2 + lax.axis_index("c")

    @pl.when(wid == 0)
    def _():
        lane = lax.iota(jnp.int32, _U)
        lanef = lane.astype(jnp.float32)

        pltpu.sync_copy(tgt_hbm, tgt_v)
        pltpu.sync_copy(tl_hbm, tl_v)
        pltpu.sync_copy(lse_hbm, lse_v)
        tgt_vec = tgt_v[...]
        tl_vec = tl_v[...]

        def build(i, _):
            r = (lane + i * _U) * _D
            idx[pl.ds(i * _U, _U)] = r + tgt_vec
            idx[pl.ds(_T * _U + i * _U, _U)] = r
            return 0

        lax.fori_loop(0, _T, build, 0, unroll=8)

        pltpu.async_copy(logits_hbm.at[idx], raw, sem).wait()

        def row(t, carry):
            la_p, st_p, tot_p, bl_p = carry
            lse_row = lse_v[pl.ds(t * _U, _U)]
            g_row = raw[pl.ds(t * _U, _U)] - lse_row
            bl_row = raw[pl.ds(_T * _U + t * _U, _U)] - lse_row

            gi = plsc.cumsum(g_row)
            G = gi - g_row
            fl = la_p + bl_p
            fl0 = jnp.where(lane == 0, 0.0, fl)
            v = fl0 - G
            pm = plsc.cummax(v)
            la = G + pm
            rec = v >= pm
            jidx = jnp.where(rec, lane, -1)
            js = plsc.cummax(jidx)
            st_s[...] = st_p
            tot_s[...] = tot_p
            stg = plsc.load_gather(st_s, [js])
            totg = plsc.load_gather(tot_s, [js])
            tf = t.astype(jnp.float32)
            j0 = js == 0
            jsf = js.astype(jnp.float32)
            st = jnp.where(j0, tf, stg)
            tot = jnp.where(j0, 1.0, totg + 1.0) + (lanef - jsf)
            laebuf[t] = la + bl_row
            stbuf[t] = st
            totbuf[t] = tot
            return (la, st, tot, bl_row)

        init = (
            jnp.full((_U,), _NEG, jnp.float32),
            jnp.zeros((_U,), jnp.float32),
            jnp.zeros((_U,), jnp.float32),
            jnp.zeros((_U,), jnp.float32),
        )
        lax.fori_loop(0, _T, row, init, unroll=8)

        for c in range(_T // _U):
            rowidx = lane + c * _U
            stage_la[pl.ds(c * _U, _U)] = plsc.load_gather(laebuf, [rowidx, tl_vec])
            stage_st[pl.ds(c * _U, _U)] = plsc.load_gather(stbuf, [rowidx, tl_vec])
            stage_tot[pl.ds(c * _U, _U)] = plsc.load_gather(totbuf, [rowidx, tl_vec]) + 1.0

        pltpu.sync_copy(stage_la, o_la)
        pltpu.sync_copy(stage_st, o_st)
        pltpu.sync_copy(stage_tot, o_tot)


def kernel(logits, targets, logit_lens, target_lens):
    lse = _tc_lse(logits).reshape(_T * _U)
    logits_flat = logits.reshape(_T * _U * _D)
    tgt = targets.reshape(_U).astype(jnp.int32)
    tl16 = jnp.broadcast_to(target_lens.astype(jnp.int32), (_U,))
    la_each, st_each, tot_each = _sc_gather_dp(logits_flat, lse, tgt, tl16)
    return (la_each[_T - 1], la_each, st_each, tot_each)
